# baseline (speedup 1.0000x reference)
import functools

import jax
import jax.numpy as jnp
from jax import lax
from jax.experimental import pallas as pl
from jax.experimental.pallas import tpu as pltpu
from jax.experimental.pallas import tpu_sc as plsc

_GI, _GJ = 64, 128
_N = _GI * _GJ
_D = 32
_NL = 10
_NLP = 128
_BM = 1024
_RG = 128
_NC = 1024
_BIG = 3.0e38
_SC_CORES = 2
_SC_SUBCORES = 16


def _argmin_rows(xg, wtn, w2):
    rg = xg.shape[0]
    m = jnp.full((rg, _GJ), _BIG, dtype=jnp.float32)
    c = jnp.zeros((rg, _GJ), dtype=jnp.int32)
    for ch in range(_N // _NC):
        kc = jnp.dot(xg, wtn[:, ch * _NC:(ch + 1) * _NC],
                     preferred_element_type=jnp.float32)
        for j in range(_NC // _GJ):
            jj = ch * (_NC // _GJ) + j
            v = kc[:, j * _GJ:(j + 1) * _GJ] + w2[:, jj * _GJ:(jj + 1) * _GJ]
            upd = v < m
            c = jnp.where(upd, jj, c)
            m = jnp.minimum(m, v)
    rowmin = jnp.min(m, axis=1, keepdims=True)
    lane = lax.broadcasted_iota(jnp.int32, (rg, _GJ), 1)
    flat = c * _GJ + lane
    cand = jnp.where(m <= rowmin, flat, _N)
    return jnp.min(cand, axis=1, keepdims=True)


def _build_table(lp):
    gr = lax.broadcasted_iota(jnp.int32, (_GI * _NL, _GI * _NL), 0)
    gc = lax.broadcasted_iota(jnp.int32, (_GI * _NL, _GI * _NL), 1)
    di = (gr // _NL - gc // _NL).astype(jnp.float32)
    gmat = jnp.exp(-0.5 * di * di) * (gr % _NL == gc % _NL).astype(jnp.float32)
    x1 = jnp.dot(lp, gmat, preferred_element_type=jnp.float32)

    br = lax.broadcasted_iota(jnp.int32, (_GJ, _GJ), 0)
    bc = lax.broadcasted_iota(jnp.int32, (_GJ, _GJ), 1)
    dj = (br - bc).astype(jnp.float32)
    bj = jnp.exp(-0.5 * dj * dj)
    x2 = jnp.dot(bj, x1, preferred_element_type=jnp.float32)

    er = lax.broadcasted_iota(jnp.int32, (_GI, _GI * _NL), 0)
    ec = lax.broadcasted_iota(jnp.int32, (_GI, _GI * _NL), 1)
    de = (er - ec // _NL).astype(jnp.float32)
    bie = jnp.exp(-0.5 * de * de)
    nae = jnp.sum(bie, axis=0, keepdims=True)
    nb = jnp.sum(bj, axis=1, keepdims=True)
    return x2 / (nae * nb)


def _som_bmu_block(x_ref, wtn_ref, lp_ref, idx_ref, tbl_ref, w2_ref):
    xb = x_ref[...]
    wtn = wtn_ref[...]

    @pl.when(pl.program_id(0) == 0)
    def _():
        w2_ref[...] = 0.25 * jnp.sum(wtn * wtn, axis=0, keepdims=True)
        tbl_ref[...] = _build_table(lp_ref[...])

    w2 = w2_ref[...]

    bmu = jnp.concatenate(
        [_argmin_rows(xb[r * _RG:(r + 1) * _RG], wtn, w2)
         for r in range(_BM // _RG)], axis=0)

    idx_ref[...] = (bmu % _GJ) * _GI + bmu // _GJ


def _sc_gather(tbl16, idxf):
    nw = _SC_CORES * _SC_SUBCORES
    per = idxf.shape[0] // nw
    mesh = plsc.VectorSubcoreMesh(core_axis_name="c", subcore_axis_name="s")

    @functools.partial(
        pl.kernel, mesh=mesh,
        out_type=jax.ShapeDtypeStruct((idxf.shape[0], _NLP), jnp.float32),
        scratch_types=[
            pltpu.VMEM((per,), jnp.int32),
            pltpu.VMEM((per, _NLP), jnp.float32),
            pltpu.SemaphoreType.DMA,
        ],
    )
    def k(tbl_hbm, idx_hbm, out_hbm, idx_v, rows_v, sem):
        wid = lax.axis_index("s") * _SC_CORES + lax.axis_index("c")
        base = wid * per
        pltpu.sync_copy(idx_hbm.at[pl.ds(base, per)], idx_v)
        pltpu.async_copy(tbl_hbm.at[idx_v], rows_v, sem).wait()
        pltpu.sync_copy(rows_v, out_hbm.at[pl.ds(base, per)])

    return k(tbl16, idxf)


@jax.jit
def kernel(x, W, L):
    B = x.shape[0]
    x = x.reshape(B, -1)
    wt = (-2.0 * W).T
    lp = L.reshape(_GI, _GJ, _NL).transpose(1, 0, 2).reshape(_GJ, _GI * _NL)
    grid = (B // _BM,)
    idx, tblraw = pl.pallas_call(
        _som_bmu_block,
        grid=grid,
        in_specs=[
            pl.BlockSpec((_BM, _D), lambda i: (i, 0)),
            pl.BlockSpec((_D, _N), lambda i: (0, 0)),
            pl.BlockSpec((_GJ, _GI * _NL), lambda i: (0, 0)),
        ],
        out_specs=[
            pl.BlockSpec((_BM, 1), lambda i: (i, 0)),
            pl.BlockSpec((_GJ, _GI * _NL), lambda i: (0, 0)),
        ],
        out_shape=[
            jax.ShapeDtypeStruct((B, 1), jnp.int32),
            jax.ShapeDtypeStruct((_GJ, _GI * _NL), jnp.float32),
        ],
        scratch_shapes=[pltpu.VMEM((1, _N), jnp.float32)],
        compiler_params=pltpu.CompilerParams(
            dimension_semantics=("arbitrary",),
        ),
    )(x, wt, lp)
    tbl16 = jnp.pad(tblraw.reshape(_N, _NL), ((0, 0), (0, _NLP - _NL)))
    out16 = _sc_gather(tbl16, idx.reshape(B))
    return out16[:, :_NL]


# speedup vs baseline: 1.0164x; 1.0164x over previous
import functools

import jax
import jax.numpy as jnp
from jax import lax
from jax.experimental import pallas as pl
from jax.experimental.pallas import tpu as pltpu
from jax.experimental.pallas import tpu_sc as plsc

_GI, _GJ = 64, 128
_N = _GI * _GJ
_D = 32
_NL = 10
_NLP = 128
_BM = 1024
_RG = 128
_NC = 1024
_BIG = 3.0e38
_SC_CORES = 2
_SC_SUBCORES = 16


def _argmin_rows(xg, wtn, w2):
    rg = xg.shape[0]
    m = jnp.full((rg, _GJ), _BIG, dtype=jnp.float32)
    c = jnp.zeros((rg, _GJ), dtype=jnp.int32)
    for ch in range(_N // _NC):
        kc = jnp.dot(xg, wtn[:, ch * _NC:(ch + 1) * _NC],
                     preferred_element_type=jnp.float32)
        for j in range(_NC // _GJ):
            jj = ch * (_NC // _GJ) + j
            v = kc[:, j * _GJ:(j + 1) * _GJ] + w2[:, jj * _GJ:(jj + 1) * _GJ]
            upd = v < m
            c = jnp.where(upd, jj, c)
            m = jnp.minimum(m, v)
    rowmin = jnp.min(m, axis=1, keepdims=True)
    lane = lax.broadcasted_iota(jnp.int32, (rg, _GJ), 1)
    flat = c * _GJ + lane
    cand = jnp.where(m <= rowmin, flat, _N)
    return jnp.min(cand, axis=1, keepdims=True)


def _build_table(lp):
    gr = lax.broadcasted_iota(jnp.int32, (_GI * _NL, _GI * _NL), 0)
    gc = lax.broadcasted_iota(jnp.int32, (_GI * _NL, _GI * _NL), 1)
    di = (gr // _NL - gc // _NL).astype(jnp.float32)
    gmat = jnp.exp(-0.5 * di * di) * (gr % _NL == gc % _NL).astype(jnp.float32)
    x1 = jnp.dot(lp, gmat, preferred_element_type=jnp.float32)

    br = lax.broadcasted_iota(jnp.int32, (_GJ, _GJ), 0)
    bc = lax.broadcasted_iota(jnp.int32, (_GJ, _GJ), 1)
    dj = (br - bc).astype(jnp.float32)
    bj = jnp.exp(-0.5 * dj * dj)
    x2 = jnp.dot(bj, x1, preferred_element_type=jnp.float32)

    er = lax.broadcasted_iota(jnp.int32, (_GI, _GI * _NL), 0)
    ec = lax.broadcasted_iota(jnp.int32, (_GI, _GI * _NL), 1)
    de = (er - ec // _NL).astype(jnp.float32)
    bie = jnp.exp(-0.5 * de * de)
    nae = jnp.sum(bie, axis=0, keepdims=True)
    nb = jnp.sum(bj, axis=1, keepdims=True)
    return x2 / (nae * nb)


def _som_bmu_block(x_ref, wtn_ref, lp_ref, idx_ref, tbl_ref, w2_ref):
    xb = x_ref[...]
    wtn = wtn_ref[...]

    @pl.when(pl.program_id(0) == 0)
    def _():
        w2_ref[...] = 0.25 * jnp.sum(wtn * wtn, axis=0, keepdims=True)
        tbl_ref[...] = _build_table(lp_ref[...])

    w2 = w2_ref[...]

    bmu = jnp.concatenate(
        [_argmin_rows(xb[r * _RG:(r + 1) * _RG], wtn, w2)
         for r in range(_BM // _RG)], axis=0)

    idx_ref[...] = ((bmu % _GJ) * _GI + bmu // _GJ)[:, 0]


def _sc_gather(tbl16, idxf):
    nw = _SC_CORES * _SC_SUBCORES
    per = idxf.shape[0] // nw
    mesh = plsc.VectorSubcoreMesh(core_axis_name="c", subcore_axis_name="s")

    @functools.partial(
        pl.kernel, mesh=mesh,
        out_type=jax.ShapeDtypeStruct((idxf.shape[0], _NLP), jnp.float32),
        scratch_types=[
            pltpu.VMEM((per,), jnp.int32),
            pltpu.VMEM((per, _NLP), jnp.float32),
            pltpu.SemaphoreType.DMA,
        ],
    )
    def k(tbl_hbm, idx_hbm, out_hbm, idx_v, rows_v, sem):
        wid = lax.axis_index("s") * _SC_CORES + lax.axis_index("c")
        base = wid * per
        pltpu.sync_copy(idx_hbm.at[pl.ds(base, per)], idx_v)
        pltpu.async_copy(tbl_hbm.at[idx_v], rows_v, sem).wait()
        pltpu.sync_copy(rows_v, out_hbm.at[pl.ds(base, per)])

    return k(tbl16, idxf)


@jax.jit
def kernel(x, W, L):
    B = x.shape[0]
    x = x.reshape(B, -1)
    wt = (-2.0 * W).T
    lp = L.reshape(_GI, _GJ, _NL).transpose(1, 0, 2).reshape(_GJ, _GI * _NL)
    grid = (B // _BM,)
    idx, tblraw = pl.pallas_call(
        _som_bmu_block,
        grid=grid,
        in_specs=[
            pl.BlockSpec((_BM, _D), lambda i: (i, 0)),
            pl.BlockSpec((_D, _N), lambda i: (0, 0)),
            pl.BlockSpec((_GJ, _GI * _NL), lambda i: (0, 0)),
        ],
        out_specs=[
            pl.BlockSpec((_BM,), lambda i: (i,)),
            pl.BlockSpec((_GJ, _GI * _NL), lambda i: (0, 0)),
        ],
        out_shape=[
            jax.ShapeDtypeStruct((B,), jnp.int32),
            jax.ShapeDtypeStruct((_GJ, _GI * _NL), jnp.float32),
        ],
        scratch_shapes=[pltpu.VMEM((1, _N), jnp.float32)],
        compiler_params=pltpu.CompilerParams(
            dimension_semantics=("arbitrary",),
        ),
    )(x, wt, lp)
    tbl16 = jnp.pad(tblraw.reshape(_N, _NL), ((0, 0), (0, _NLP - _NL)))
    out16 = _sc_gather(tbl16, idx)
    return out16[:, :_NL]


# speedup vs baseline: 1.0223x; 1.0057x over previous
import functools

import jax
import jax.numpy as jnp
from jax import lax
from jax.experimental import pallas as pl
from jax.experimental.pallas import tpu as pltpu
from jax.experimental.pallas import tpu_sc as plsc

_GI, _GJ = 64, 128
_N = _GI * _GJ
_D = 32
_NL = 10
_NLP = 128
_BM = 2048
_RG = 128
_NC = 1024
_BIG = 3.0e38
_SC_CORES = 2
_SC_SUBCORES = 16


def _argmin_rows(xg, wtn, w2):
    rg = xg.shape[0]
    m = jnp.full((rg, _GJ), _BIG, dtype=jnp.float32)
    c = jnp.zeros((rg, _GJ), dtype=jnp.int32)
    for ch in range(_N // _NC):
        kc = jnp.dot(xg, wtn[:, ch * _NC:(ch + 1) * _NC],
                     preferred_element_type=jnp.float32)
        for j in range(_NC // _GJ):
            jj = ch * (_NC // _GJ) + j
            v = kc[:, j * _GJ:(j + 1) * _GJ] + w2[:, jj * _GJ:(jj + 1) * _GJ]
            upd = v < m
            c = jnp.where(upd, jj, c)
            m = jnp.minimum(m, v)
    rowmin = jnp.min(m, axis=1, keepdims=True)
    lane = lax.broadcasted_iota(jnp.int32, (rg, _GJ), 1)
    flat = c * _GJ + lane
    cand = jnp.where(m <= rowmin, flat, _N)
    return jnp.min(cand, axis=1, keepdims=True)


def _build_table(lp):
    gr = lax.broadcasted_iota(jnp.int32, (_GI * _NL, _GI * _NL), 0)
    gc = lax.broadcasted_iota(jnp.int32, (_GI * _NL, _GI * _NL), 1)
    di = (gr // _NL - gc // _NL).astype(jnp.float32)
    gmat = jnp.exp(-0.5 * di * di) * (gr % _NL == gc % _NL).astype(jnp.float32)
    x1 = jnp.dot(lp, gmat, preferred_element_type=jnp.float32)

    br = lax.broadcasted_iota(jnp.int32, (_GJ, _GJ), 0)
    bc = lax.broadcasted_iota(jnp.int32, (_GJ, _GJ), 1)
    dj = (br - bc).astype(jnp.float32)
    bj = jnp.exp(-0.5 * dj * dj)
    x2 = jnp.dot(bj, x1, preferred_element_type=jnp.float32)

    er = lax.broadcasted_iota(jnp.int32, (_GI, _GI * _NL), 0)
    ec = lax.broadcasted_iota(jnp.int32, (_GI, _GI * _NL), 1)
    de = (er - ec // _NL).astype(jnp.float32)
    bie = jnp.exp(-0.5 * de * de)
    nae = jnp.sum(bie, axis=0, keepdims=True)
    nb = jnp.sum(bj, axis=1, keepdims=True)
    return x2 / (nae * nb)


def _som_bmu_block(x_ref, wtn_ref, lp_ref, idx_ref, tbl_ref, w2_ref):
    xb = x_ref[...]
    wtn = wtn_ref[...]

    @pl.when(pl.program_id(0) == 0)
    def _():
        w2_ref[...] = 0.25 * jnp.sum(wtn * wtn, axis=0, keepdims=True)
        tbl_ref[...] = _build_table(lp_ref[...])

    w2 = w2_ref[...]

    bmu = jnp.concatenate(
        [_argmin_rows(xb[r * _RG:(r + 1) * _RG], wtn, w2)
         for r in range(_BM // _RG)], axis=0)

    idx_ref[...] = ((bmu % _GJ) * _GI + bmu // _GJ)[:, 0]


def _sc_gather(tbl16, idxf):
    nw = _SC_CORES * _SC_SUBCORES
    per = idxf.shape[0] // nw
    mesh = plsc.VectorSubcoreMesh(core_axis_name="c", subcore_axis_name="s")

    @functools.partial(
        pl.kernel, mesh=mesh,
        out_type=jax.ShapeDtypeStruct((idxf.shape[0], _NLP), jnp.float32),
        scratch_types=[
            pltpu.VMEM((per,), jnp.int32),
            pltpu.VMEM((per, _NLP), jnp.float32),
            pltpu.SemaphoreType.DMA,
        ],
    )
    def k(tbl_hbm, idx_hbm, out_hbm, idx_v, rows_v, sem):
        wid = lax.axis_index("s") * _SC_CORES + lax.axis_index("c")
        base = wid * per
        pltpu.sync_copy(idx_hbm.at[pl.ds(base, per)], idx_v)
        pltpu.async_copy(tbl_hbm.at[idx_v], rows_v, sem).wait()
        pltpu.sync_copy(rows_v, out_hbm.at[pl.ds(base, per)])

    return k(tbl16, idxf)


@jax.jit
def kernel(x, W, L):
    B = x.shape[0]
    x = x.reshape(B, -1)
    wt = (-2.0 * W).T
    lp = L.reshape(_GI, _GJ, _NL).transpose(1, 0, 2).reshape(_GJ, _GI * _NL)
    grid = (B // _BM,)
    idx, tblraw = pl.pallas_call(
        _som_bmu_block,
        grid=grid,
        in_specs=[
            pl.BlockSpec((_BM, _D), lambda i: (i, 0)),
            pl.BlockSpec((_D, _N), lambda i: (0, 0)),
            pl.BlockSpec((_GJ, _GI * _NL), lambda i: (0, 0)),
        ],
        out_specs=[
            pl.BlockSpec((_BM,), lambda i: (i,)),
            pl.BlockSpec((_GJ, _GI * _NL), lambda i: (0, 0)),
        ],
        out_shape=[
            jax.ShapeDtypeStruct((B,), jnp.int32),
            jax.ShapeDtypeStruct((_GJ, _GI * _NL), jnp.float32),
        ],
        scratch_shapes=[pltpu.VMEM((1, _N), jnp.float32)],
        compiler_params=pltpu.CompilerParams(
            dimension_semantics=("arbitrary",),
        ),
    )(x, wt, lp)
    tbl16 = jnp.pad(tblraw.reshape(_N, _NL), ((0, 0), (0, _NLP - _NL)))
    out16 = _sc_gather(tbl16, idx)
    return out16[:, :_NL]
